# baseline (speedup 1.0000x reference)
import jax
import jax.numpy as jnp
from jax import lax
from jax.experimental import pallas as pl

C, W, H = 20, 1024, 1024


def _zeros_kernel(out_ref):
    out_ref[...] = jnp.zeros_like(out_ref)


def _make_zero_grid():
    return pl.pallas_call(
        _zeros_kernel,
        out_shape=jax.ShapeDtypeStruct((C, W, H), jnp.float32),
        grid=(C,),
        out_specs=pl.BlockSpec((1, W, H), lambda i: (i, 0, 0)),
    )()


def _sorted_winner_scatter(grid, ch, x, y, vals):
    key = ((ch * W + x) * H + y).astype(jnp.int32).reshape(-1)
    val = vals.astype(jnp.float32).reshape(-1)
    sk, sv = lax.sort_key_val(key, val, is_stable=False)
    last = jnp.concatenate([sk[:-1] != sk[1:], jnp.ones((1,), bool)])
    flat = grid.reshape(-1)
    skw = jnp.where(last, sk, C * W * H)
    flat = flat.at[skw].set(sv, mode="drop", unique_indices=True)
    return flat.reshape(C, W, H)


def kernel(body_xy, body_feat, body_dyn, joint_A_xy, joint_B_xy, joint_feat, joint_dyn):
    grid = _make_zero_grid()

    bx, by = body_xy[:, 0], body_xy[:, 1]
    bd = body_dyn.astype(jnp.int32)
    ch_b = (bd * 5)[:, None] + jnp.arange(5)[None, :]
    grid = _sorted_winner_scatter(
        grid, ch_b, bx[:, None] + 0 * ch_b, by[:, None] + 0 * ch_b, body_feat)
    grid = grid.at[18 + bd, bx, by].set(1.0, mode="drop")

    ax, ay = joint_A_xy[:, 0], joint_A_xy[:, 1]
    jbx, jby = joint_B_xy[:, 0], joint_B_xy[:, 1]
    jd = joint_dyn.astype(jnp.int32)

    ch_a = (10 + jd * 4)[:, None] + jnp.arange(2)[None, :]
    grid = _sorted_winner_scatter(
        grid, ch_a, ax[:, None] + 0 * ch_a, ay[:, None] + 0 * ch_a, joint_feat)
    ch_bj = ch_a + 2
    grid = _sorted_winner_scatter(
        grid, ch_bj, jbx[:, None] + 0 * ch_bj, jby[:, None] + 0 * ch_bj, joint_feat)

    grid = grid.at[18 + jd, ax, ay].set(1.0, mode="drop")
    grid = grid.at[18 + jd, jbx, jby].set(1.0, mode="drop")

    return grid[None]


# speedup vs baseline: 1.2152x; 1.2152x over previous
import functools

import jax
import jax.numpy as jnp
from jax import lax
from jax.experimental import pallas as pl
from jax.experimental.pallas import tpu as pltpu
from jax.experimental.pallas import tpu_sc as plsc

C, W, H = 20, 1024, 1024
GRID = C * W * H
PADW = 16384
OUT_WORDS = GRID + PADW
SPLIT = 12 * W * H

N_ENTRIES = 500000 + 200000 + 200000 + 300000
PER_W = 75264
M_PAD = PER_W * 16
NBUF = 4
NBLOCKS = PER_W // 128
NOUT = NBLOCKS // NBUF
ZBW = 16384

_mesh = plsc.VectorSubcoreMesh(core_axis_name="c", subcore_axis_name="s")


def _sc_scatter_body(addr_h, val_h, zeros_h, out_h, zbuf, abuf, vbuf,
                     asc, vsc, zsem, isem, osem):
    c = lax.axis_index("c")
    s = lax.axis_index("s")
    w_off = s * PER_W
    lo = c * SPLIT
    hi = jnp.where(c == 0, SPLIT, GRID)

    pltpu.async_copy(zeros_h, zbuf, zsem)
    pltpu.make_async_copy(zeros_h, zbuf, zsem).wait()

    @pl.when(c == 0)
    def _():
        for k in range(48):
            pltpu.async_copy(
                zbuf, out_h.at[pl.ds(s * (48 * ZBW) + k * ZBW, ZBW)], zsem)

    @pl.when(c == 1)
    def _():
        for k in range(32):
            pltpu.async_copy(
                zbuf, out_h.at[pl.ds(SPLIT + s * (32 * ZBW) + k * ZBW, ZBW)], zsem)

    for k in range(NBUF):
        pltpu.async_copy(addr_h.at[pl.ds(w_off + k * 128, 128)], abuf.at[k],
                         isem.at[k])
        pltpu.async_copy(val_h.at[pl.ds(w_off + k * 128, 128)], vbuf.at[k],
                         isem.at[k])

    @pl.when(c == 0)
    def _():
        for k in range(48):
            pltpu.make_async_copy(zbuf, out_h.at[pl.ds(0, ZBW)], zsem).wait()

    @pl.when(c == 1)
    def _():
        for k in range(32):
            pltpu.make_async_copy(zbuf, out_h.at[pl.ds(0, ZBW)], zsem).wait()

    plsc.subcore_barrier()

    iota16 = lax.broadcasted_iota(jnp.int32, (16,), 0)

    def outer(i, carry):
        for k in range(NBUF):
            blk = i * NBUF + k
            @pl.when(i > 0)
            def _():
                pltpu.make_async_copy(vsc.at[k], out_h.at[asc.at[k]],
                                      osem.at[k]).wait()

            pltpu.make_async_copy(addr_h.at[pl.ds(0, 128)], abuf.at[k],
                                  isem.at[k]).wait()
            pltpu.make_async_copy(val_h.at[pl.ds(0, 128)], vbuf.at[k],
                                  isem.at[k]).wait()

            off = w_off + blk * 128
            for j in range(8):
                a = abuf[k, 16 * j:16 * j + 16]
                owned = (a >= lo) & (a < hi)
                dump = GRID + ((off + 16 * j) & (PADW - 1)) + iota16
                asc[k, 16 * j:16 * j + 16] = jnp.where(owned, a, dump)
                vsc[k, 16 * j:16 * j + 16] = vbuf[k, 16 * j:16 * j + 16]

            pltpu.async_copy(vsc.at[k], out_h.at[asc.at[k]], osem.at[k])

            nblk = blk + NBUF

            @pl.when(nblk < NBLOCKS)
            def _():
                pltpu.async_copy(addr_h.at[pl.ds(w_off + nblk * 128, 128)],
                                 abuf.at[k], isem.at[k])
                pltpu.async_copy(val_h.at[pl.ds(w_off + nblk * 128, 128)],
                                 vbuf.at[k], isem.at[k])
        return carry

    lax.fori_loop(0, NOUT, outer, 0)

    for k in range(NBUF):
        pltpu.make_async_copy(vsc.at[k], out_h.at[asc.at[k]],
                              osem.at[k]).wait()


@functools.partial(jax.jit, static_argnums=())
def _sc_scatter(addr, val, zeros):
    return pl.kernel(
        _sc_scatter_body,
        out_type=jax.ShapeDtypeStruct((OUT_WORDS,), jnp.float32),
        mesh=_mesh,
        scratch_types=[
            pltpu.VMEM((ZBW,), jnp.float32),
            pltpu.VMEM((NBUF, 128), jnp.int32),
            pltpu.VMEM((NBUF, 128), jnp.float32),
            pltpu.VMEM((NBUF, 128), jnp.int32),
            pltpu.VMEM((NBUF, 128), jnp.float32),
            pltpu.SemaphoreType.DMA,
            pltpu.SemaphoreType.DMA((NBUF,)),
            pltpu.SemaphoreType.DMA((NBUF,)),
        ],
    )(addr, val, zeros)


def _winner_entries(ch, x, y, vals):
    key = ((ch * W + x) * H + y).astype(jnp.int32).reshape(-1)
    val = vals.astype(jnp.float32).reshape(-1)
    sk, sv = lax.sort_key_val(key, val, is_stable=False)
    last = jnp.concatenate([sk[:-1] != sk[1:], jnp.ones((1,), bool)])
    n = sk.shape[0]
    dump = GRID + (jnp.arange(n, dtype=jnp.int32) & (PADW - 1))
    return jnp.where(last, sk, dump), sv


def kernel(body_xy, body_feat, body_dyn, joint_A_xy, joint_B_xy, joint_feat, joint_dyn):
    bx, by = body_xy[:, 0], body_xy[:, 1]
    bd = body_dyn.astype(jnp.int32)
    ax, ay = joint_A_xy[:, 0], joint_A_xy[:, 1]
    jbx, jby = joint_B_xy[:, 0], joint_B_xy[:, 1]
    jd = joint_dyn.astype(jnp.int32)

    ch_b = (bd * 5)[:, None] + jnp.arange(5)[None, :]
    a5, v5 = _winner_entries(ch_b, bx[:, None] + 0 * ch_b,
                             by[:, None] + 0 * ch_b, body_feat)
    ch_a = (10 + jd * 4)[:, None] + jnp.arange(2)[None, :]
    aA, vA = _winner_entries(ch_a, ax[:, None] + 0 * ch_a,
                             ay[:, None] + 0 * ch_a, joint_feat)
    ch_bj = ch_a + 2
    aB, vB = _winner_entries(ch_bj, jbx[:, None] + 0 * ch_bj,
                             jby[:, None] + 0 * ch_bj, joint_feat)

    pres = jnp.concatenate([
        ((18 + bd) * W + bx) * H + by,
        ((18 + jd) * W + ax) * H + ay,
        ((18 + jd) * W + jbx) * H + jby,
    ]).astype(jnp.int32)

    npad = M_PAD - N_ENTRIES
    addr = jnp.concatenate([
        a5, aA, aB, pres,
        GRID + (jnp.arange(npad, dtype=jnp.int32) & (PADW - 1)),
    ])
    val = jnp.concatenate([
        v5, vA, vB,
        jnp.ones((300000,), jnp.float32),
        jnp.zeros((npad,), jnp.float32),
    ])

    out = _sc_scatter(addr, val, jnp.zeros((ZBW,), jnp.float32))
    return out[:GRID].reshape(1, C, W, H)


# speedup vs baseline: 4.2458x; 3.4939x over previous
import functools

import jax
import jax.numpy as jnp
from jax import lax
from jax.experimental import pallas as pl
from jax.experimental.pallas import tpu as pltpu
from jax.experimental.pallas import tpu_sc as plsc

C, W, H = 20, 1024, 1024
GRID = C * W * H
HALF = GRID // 2
NWIN = 8
WLEN = HALF // NWIN
WPW = WLEN // 16
DUMP = WLEN
NBUF = 4
ZBW = 16384
APAD = 8576

_mesh = plsc.VectorSubcoreMesh(core_axis_name="c", subcore_axis_name="s")

_ARRAYS = ((0, 0, True), (1, 1, True), (2, 1, True), (3, 1, False))


def _sc_scatter_body(a5_h, v5_h, aA_h, vA_h, aB_h, vB_h, ap_h, bounds_h,
                     zeros_h, out_h, sbuf, zbuf, bbuf, abuf, vbuf, asc, vsc,
                     zsem, bsem, isem, osem):
    c = lax.axis_index("c")
    s = lax.axis_index("s")
    addr_refs = (a5_h, aA_h, aB_h, ap_h)
    val_refs = (v5_h, vA_h, vB_h, None)

    pltpu.async_copy(zeros_h, zbuf, zsem)
    pltpu.async_copy(bounds_h, bbuf, bsem)
    pltpu.make_async_copy(zeros_h, zbuf, zsem).wait()
    pltpu.make_async_copy(bounds_h, bbuf, bsem).wait()

    iota16 = lax.broadcasted_iota(jnp.int32, (16,), 0)
    one16 = jnp.ones((16,), jnp.float32)

    def wbody(w, carry):
        w0 = c * HALF + w * WLEN

        for k in range(5):
            pltpu.async_copy(zbuf, sbuf.at[pl.ds(s * WPW + k * ZBW, ZBW)], zsem)
        for k in range(5):
            pltpu.make_async_copy(zbuf, sbuf.at[pl.ds(0, ZBW)], zsem).wait()
        plsc.subcore_barrier()

        for (aid, owner, has_val) in _ARRAYS:
            a_h = addr_refs[aid]
            v_h = val_refs[aid]
            arow = bbuf[16 * aid:16 * aid + 16]
            lo = jnp.int32(0)
            hi = jnp.int32(0)
            for l in range(8):
                lo = jnp.where(w == l, arow[l], lo)
                hi = jnp.where(w == l, arow[8 + l], hi)
            b0 = lo // 128
            bend = (hi + 127) // 128
            per_w = (bend - b0 + 15) // 16
            per_w4 = ((per_w + 3) // 4) * 4
            myb0 = b0 + s * per_w4

            @pl.when((c == owner) & (per_w4 > 0))
            def _():
                for k in range(NBUF):
                    pltpu.async_copy(
                        a_h.at[pl.ds((myb0 + k) * 128, 128)], abuf.at[k],
                        isem.at[k])
                    if has_val:
                        pltpu.async_copy(
                            v_h.at[pl.ds((myb0 + k) * 128, 128)], vbuf.at[k],
                            isem.at[k])

                def tbody(t, tc):
                    for k in range(NBUF):
                        b = myb0 + t * NBUF + k

                        @pl.when(t > 0)
                        def _():
                            pltpu.make_async_copy(
                                vsc.at[k], sbuf.at[asc.at[k]],
                                osem.at[k]).wait()

                        pltpu.make_async_copy(
                            a_h.at[pl.ds(0, 128)], abuf.at[k],
                            isem.at[k]).wait()
                        if has_val:
                            pltpu.make_async_copy(
                                v_h.at[pl.ds(0, 128)], vbuf.at[k],
                                isem.at[k]).wait()

                        for j in range(8):
                            a = abuf[k, 16 * j:16 * j + 16]
                            pv = b * 128 + 16 * j + iota16
                            m = ((pv >= lo) & (pv < hi)
                                 & (a >= w0) & (a < w0 + WLEN))
                            asc[k, 16 * j:16 * j + 16] = jnp.where(
                                m, a - w0, DUMP)
                            if has_val:
                                vsc[k, 16 * j:16 * j + 16] = \
                                    vbuf[k, 16 * j:16 * j + 16]
                            else:
                                vsc[k, 16 * j:16 * j + 16] = one16

                        pltpu.async_copy(vsc.at[k], sbuf.at[asc.at[k]],
                                         osem.at[k])

                        @pl.when(t * NBUF + k + NBUF < per_w4)
                        def _():
                            pltpu.async_copy(
                                a_h.at[pl.ds((b + NBUF) * 128, 128)],
                                abuf.at[k], isem.at[k])
                            if has_val:
                                pltpu.async_copy(
                                    v_h.at[pl.ds((b + NBUF) * 128, 128)],
                                    vbuf.at[k], isem.at[k])
                    return tc

                lax.fori_loop(0, per_w4 // NBUF, tbody, 0)

                for k in range(NBUF):
                    pltpu.make_async_copy(vsc.at[k], sbuf.at[asc.at[k]],
                                          osem.at[k]).wait()

        plsc.subcore_barrier()

        for k in range(5):
            pltpu.async_copy(sbuf.at[pl.ds(s * WPW + k * ZBW, ZBW)],
                             out_h.at[pl.ds(w0 + s * WPW + k * ZBW, ZBW)],
                             zsem)
        for k in range(5):
            pltpu.make_async_copy(sbuf.at[pl.ds(0, ZBW)],
                                  out_h.at[pl.ds(w0, ZBW)], zsem).wait()
        return carry

    lax.fori_loop(0, NWIN, wbody, 0)


@functools.partial(jax.jit, static_argnums=())
def _sc_scatter(a5, v5, aA, vA, aB, vB, ap, bounds, zeros):
    return pl.kernel(
        _sc_scatter_body,
        out_type=jax.ShapeDtypeStruct((GRID,), jnp.float32),
        mesh=_mesh,
        scratch_types=[
            pltpu.VMEM_SHARED((WLEN + 16,), jnp.float32),
            pltpu.VMEM((ZBW,), jnp.float32),
            pltpu.VMEM((64,), jnp.int32),
            pltpu.VMEM((NBUF, 128), jnp.int32),
            pltpu.VMEM((NBUF, 128), jnp.float32),
            pltpu.VMEM((NBUF, 128), jnp.int32),
            pltpu.VMEM((NBUF, 128), jnp.float32),
            pltpu.SemaphoreType.DMA,
            pltpu.SemaphoreType.DMA,
            pltpu.SemaphoreType.DMA((NBUF,)),
            pltpu.SemaphoreType.DMA((NBUF,)),
        ],
    )(a5, v5, aA, vA, aB, vB, ap, bounds, zeros)


def _winner_entries(ch, x, y, vals):
    key = ((ch * W + x) * H + y).astype(jnp.int32).reshape(-1)
    val = vals.astype(jnp.float32).reshape(-1)
    sk, sv = lax.sort_key_val(key, val, is_stable=False)
    n = sk.shape[0]
    last = jnp.concatenate([sk[:-1] != sk[1:], jnp.ones((1,), bool)])
    li = jnp.where(last, jnp.arange(n, dtype=jnp.int32), jnp.int32(n))
    runlast = lax.cummin(li, axis=0, reverse=True)
    return sk, sv[runlast]


def _pad_to(a, n, fill):
    return jnp.concatenate([a, jnp.full((n - a.shape[0],), fill, a.dtype)])


def kernel(body_xy, body_feat, body_dyn, joint_A_xy, joint_B_xy, joint_feat, joint_dyn):
    bx, by = body_xy[:, 0], body_xy[:, 1]
    bd = body_dyn.astype(jnp.int32)
    ax, ay = joint_A_xy[:, 0], joint_A_xy[:, 1]
    jbx, jby = joint_B_xy[:, 0], joint_B_xy[:, 1]
    jd = joint_dyn.astype(jnp.int32)

    ch_b = (bd * 5)[:, None] + jnp.arange(5)[None, :]
    a5, v5 = _winner_entries(ch_b, bx[:, None] + 0 * ch_b,
                             by[:, None] + 0 * ch_b, body_feat)
    ch_a = (10 + jd * 4)[:, None] + jnp.arange(2)[None, :]
    aA, vA = _winner_entries(ch_a, ax[:, None] + 0 * ch_a,
                             ay[:, None] + 0 * ch_a, joint_feat)
    ch_bj = ch_a + 2
    aB, vB = _winner_entries(ch_bj, jbx[:, None] + 0 * ch_bj,
                             jby[:, None] + 0 * ch_bj, joint_feat)

    ap = jnp.concatenate([
        ((18 + bd) * W + bx) * H + by,
        ((18 + jd) * W + ax) * H + ay,
        ((18 + jd) * W + jbx) * H + jby,
    ]).astype(jnp.int32)

    e0 = jnp.arange(NWIN + 1, dtype=jnp.int32) * WLEN
    ss5 = jnp.searchsorted(a5, e0).astype(jnp.int32)
    ssA = jnp.searchsorted(aA, HALF + e0).astype(jnp.int32)
    ssB = jnp.searchsorted(aB, HALF + e0).astype(jnp.int32)
    np_ = ap.shape[0]
    plo = jnp.where(jnp.arange(NWIN) >= 6, 0, 0).astype(jnp.int32)
    phi = jnp.where(jnp.arange(NWIN) >= 6, np_, 0).astype(jnp.int32)
    bounds = jnp.concatenate([
        ss5[:8], ss5[1:9], ssA[:8], ssA[1:9], ssB[:8], ssB[1:9], plo, phi,
    ])

    far = jnp.int32(2 ** 29)
    out = _sc_scatter(
        _pad_to(a5, 500000 + APAD, far), _pad_to(v5, 500000 + APAD, 0.0),
        _pad_to(aA, 200000 + APAD, far), _pad_to(vA, 200000 + APAD, 0.0),
        _pad_to(aB, 200000 + APAD, far), _pad_to(vB, 200000 + APAD, 0.0),
        _pad_to(ap, 300000 + APAD, far),
        bounds, jnp.zeros((ZBW,), jnp.float32))
    return out.reshape(1, C, W, H)
